# baseline (speedup 1.0000x reference)
import functools

import jax
import jax.numpy as jnp
from jax import lax
from jax.experimental import pallas as pl
from jax.experimental.pallas import tpu as pltpu
from jax.experimental.pallas import tpu_sc as plsc

DIM = 64
LANES = 16


def _tree_sum_kernel(B, bpw, NC):
    mesh = plsc.VectorSubcoreMesh(core_axis_name="c", subcore_axis_name="s")

    @functools.partial(
        pl.kernel,
        mesh=mesh,
        compiler_params=pltpu.CompilerParams(use_tc_tiling_on_sc=False),
        out_type=jax.ShapeDtypeStruct((B, DIM), jnp.float32),
        scratch_types=[
            pltpu.VMEM((bpw,), jnp.int32),
            pltpu.VMEM((bpw,), jnp.int32),
            pltpu.VMEM((bpw,), jnp.int32),
            pltpu.VMEM((bpw,), jnp.int32),
            pltpu.VMEM((bpw, DIM), jnp.float32),
            pltpu.VMEM((bpw, DIM), jnp.float32),
            pltpu.VMEM((bpw, DIM), jnp.float32),
            pltpu.VMEM((bpw, DIM), jnp.float32),
            pltpu.SemaphoreType.DMA,
        ],
    )
    def k(idx_hbm, t0, t1, t2, t3, p0, p1, p2, out_hbm,
          i0, i1, i2, i3, buf0, buf1, buf2, buf3, sem):
        wid = lax.axis_index("s") * NC + lax.axis_index("c")
        base = wid * bpw

        pltpu.sync_copy(idx_hbm.at[pl.ds(base, bpw)], i0)
        c0 = pltpu.async_copy(t0.at[i0], buf0, sem)
        pltpu.sync_copy(p0.at[i0], i1)
        c1 = pltpu.async_copy(t1.at[i1], buf1, sem)
        pltpu.sync_copy(p1.at[i1], i2)
        c2 = pltpu.async_copy(t2.at[i2], buf2, sem)
        pltpu.sync_copy(p2.at[i2], i3)
        c3 = pltpu.async_copy(t3.at[i3], buf3, sem)
        c0.wait()
        c1.wait()
        c2.wait()
        c3.wait()

        def row(r, _):
            for j in range(DIM // LANES):
                d = pl.ds(j * LANES, LANES)
                buf0[r, d] = (buf0[r, d] + buf1[r, d]) + (buf2[r, d] + buf3[r, d])
            return 0

        lax.fori_loop(0, bpw, row, 0)
        pltpu.sync_copy(buf0, out_hbm.at[pl.ds(base, bpw)])

    return k


def kernel(idx, T0, T1, T2, T3, P0, P1, P2):
    B = idx.shape[0]
    info = plsc.get_sparse_core_info()
    NC = info.num_cores
    NW = NC * info.num_subcores
    bpw = B // NW
    k = _tree_sum_kernel(B, bpw, NC)
    return k(idx.astype(jnp.int32), T0, T1, T2, T3, P0, P1, P2)
